# Initial kernel scaffold; baseline (speedup 1.0000x reference)
import jax
import jax.numpy as jnp
from jax.experimental import pallas as pl


def kernel(x, edge_index, W1, b1, W2, b2):
    raise NotImplementedError("write your pallas kernel here")


# speedup vs baseline: 30.1091x; 30.1091x over previous
import jax
import jax.numpy as jnp
from jax import lax
from jax.experimental import pallas as pl
from jax.experimental.pallas import tpu as pltpu
from jax.experimental.pallas import tpu_sc as plsc

N_NODES = 10000
N_EDGES = 320000
IN_FEATS = 128
HIDDEN = 16
OUT_FEATS = 2

NC, NS = 2, 16
NW = NC * NS
NP = 10240
RPT = NP // NS
EPW = N_EDGES // NW
K = 80
NCHUNK = EPW // K


def _sc_mesh():
    return plsc.VectorSubcoreMesh(
        core_axis_name="c", subcore_axis_name="s",
        num_cores=NC, num_subcores=NS)


def _deg_body(dst_hbm, ones_hbm, zeros_hbm, out_hbm, dst_v, ones_v, acc_sh):
    c = lax.axis_index("c")
    s = lax.axis_index("s")
    wid = s * NC + c
    pltpu.sync_copy(zeros_hbm.at[pl.ds(s * RPT, RPT)],
                    acc_sh.at[pl.ds(s * RPT, RPT)])
    pltpu.sync_copy(dst_hbm.at[wid], dst_v)
    pltpu.sync_copy(ones_hbm, ones_v)
    plsc.subcore_barrier()

    def body(j, carry):
        pltpu.sync_copy(ones_v, acc_sh.at[dst_v.at[j]], add=True)
        return carry
    lax.fori_loop(0, NCHUNK, body, 0)

    plsc.subcore_barrier()
    pltpu.sync_copy(acc_sh.at[pl.ds(s * RPT, RPT)],
                    out_hbm.at[c].at[pl.ds(s * RPT, RPT)])


def _agg_body(rows_hbm, src_hbm, dst_hbm, zeros_hbm, out_hbm,
              src_v, dst_v, rows_v, acc_sh, sem):
    c = lax.axis_index("c")
    s = lax.axis_index("s")
    wid = s * NC + c
    pltpu.sync_copy(zeros_hbm.at[pl.ds(s * RPT, RPT)],
                    acc_sh.at[pl.ds(s * RPT, RPT)])
    pltpu.sync_copy(src_hbm.at[wid], src_v)
    pltpu.sync_copy(dst_hbm.at[wid], dst_v)
    plsc.subcore_barrier()

    def body(j, carry):
        pltpu.async_copy(rows_hbm.at[src_v.at[j]], rows_v, sem).wait()
        pltpu.sync_copy(rows_v, acc_sh.at[dst_v.at[j]], add=True)
        return carry
    lax.fori_loop(0, NCHUNK, body, 0)

    plsc.subcore_barrier()
    pltpu.sync_copy(acc_sh.at[pl.ds(s * RPT, RPT)],
                    out_hbm.at[c].at[pl.ds(s * RPT, RPT)])


_sc_params = pltpu.CompilerParams(use_tc_tiling_on_sc=False)

_deg_call = pl.kernel(
    _deg_body,
    out_type=jax.ShapeDtypeStruct((NC, NP), jnp.float32),
    mesh=_sc_mesh(),
    compiler_params=_sc_params,
    scratch_types=[
        pltpu.VMEM((NCHUNK, K), jnp.int32),
        pltpu.VMEM((K,), jnp.float32),
        pltpu.VMEM_SHARED((NP,), jnp.float32),
    ],
)

_agg_call = pl.kernel(
    _agg_body,
    out_type=jax.ShapeDtypeStruct((NC, NP, HIDDEN), jnp.float32),
    mesh=_sc_mesh(),
    compiler_params=_sc_params,
    scratch_types=[
        pltpu.VMEM((NCHUNK, K), jnp.int32),
        pltpu.VMEM((NCHUNK, K), jnp.int32),
        pltpu.VMEM((K, HIDDEN), jnp.float32),
        pltpu.VMEM_SHARED((NP, HIDDEN), jnp.float32),
        pltpu.SemaphoreType.DMA,
    ],
)


def _tc1_body(xp, w1, degp, hs, dinvb):
    deg = degp[0, :] + degp[1, :] + 1.0
    dinv = lax.rsqrt(deg)
    db = jnp.broadcast_to(dinv[:, None], (NP, HIDDEN))
    dinvb[...] = db
    h = jnp.dot(xp[...], w1[...], preferred_element_type=jnp.float32)
    hs[...] = h * db


def _tc2_body(a1p, hs, dinvb, b1, gs):
    a1 = (a1p[0] + a1p[1] + hs[...]) * dinvb[...] + b1[...]
    gs[...] = jnp.maximum(a1, 0.0) * dinvb[...]


def _tc3_body(a2p, gs, dinvb, w2, b2, out):
    a2 = (a2p[0] + a2p[1] + gs[...]) * dinvb[...]
    out[...] = jnp.dot(a2, w2[...], preferred_element_type=jnp.float32) + b2[...]


_tc1 = pl.pallas_call(
    _tc1_body,
    out_shape=[jax.ShapeDtypeStruct((NP, HIDDEN), jnp.float32),
               jax.ShapeDtypeStruct((NP, HIDDEN), jnp.float32)],
)

_tc2 = pl.pallas_call(
    _tc2_body,
    out_shape=jax.ShapeDtypeStruct((NP, HIDDEN), jnp.float32),
)

_tc3 = pl.pallas_call(
    _tc3_body,
    out_shape=jax.ShapeDtypeStruct((NP, OUT_FEATS), jnp.float32),
)


def kernel(x, edge_index, W1, b1, W2, b2):
    src = edge_index[0].astype(jnp.int32).reshape(NW, NCHUNK, K)
    dst = edge_index[1].astype(jnp.int32).reshape(NW, NCHUNK, K)
    xp = jnp.zeros((NP, IN_FEATS), jnp.float32).at[:N_NODES].set(x)
    zeros1 = jnp.zeros((NP,), jnp.float32)
    zerosH = jnp.zeros((NP, HIDDEN), jnp.float32)
    onesK = jnp.ones((K,), jnp.float32)

    degp = _deg_call(dst, onesK, zeros1)
    hs, dinvb = _tc1(xp, W1, degp)
    a1p = _agg_call(hs, src, dst, zerosH)
    gs = _tc2(a1p, hs, dinvb, b1.reshape(1, HIDDEN))
    a2p = _agg_call(gs, src, dst, zerosH)
    out = _tc3(a2p, gs, dinvb, W2, b2.reshape(1, OUT_FEATS))
    return out[:N_NODES]


# speedup vs baseline: 56.1734x; 1.8657x over previous
import jax
import jax.numpy as jnp
from jax import lax
from jax.experimental import pallas as pl
from jax.experimental.pallas import tpu as pltpu
from jax.experimental.pallas import tpu_sc as plsc

N_NODES = 10000
N_EDGES = 320000
IN_FEATS = 128
HIDDEN = 16
OUT_FEATS = 2

NC, NS = 2, 16
NW = NC * NS
NP = 10240
RPT = NP // NS
EPW = N_EDGES // NW
K = 80
NCHUNK = EPW // K


def _sc_mesh():
    return plsc.VectorSubcoreMesh(
        core_axis_name="c", subcore_axis_name="s",
        num_cores=NC, num_subcores=NS)


def _deg_body(dst_hbm, ones_hbm, zeros_hbm, out_hbm, dst_v, ones_v, acc_sh):
    c = lax.axis_index("c")
    s = lax.axis_index("s")
    wid = s * NC + c
    pltpu.sync_copy(zeros_hbm.at[pl.ds(s * RPT, RPT)],
                    acc_sh.at[pl.ds(s * RPT, RPT)])
    pltpu.sync_copy(dst_hbm.at[wid], dst_v)
    pltpu.sync_copy(ones_hbm, ones_v)
    plsc.subcore_barrier()

    def body(j, carry):
        pltpu.sync_copy(ones_v, acc_sh.at[dst_v.at[j]], add=True)
        return carry
    lax.fori_loop(0, NCHUNK, body, 0)

    plsc.subcore_barrier()
    pltpu.sync_copy(acc_sh.at[pl.ds(s * RPT, RPT)],
                    out_hbm.at[c].at[pl.ds(s * RPT, RPT)])


NBUF = 5
NGRP = NCHUNK // NBUF


def _agg_body(rows_hbm, src_hbm, dst_hbm, zeros_hbm, out_hbm,
              src_v, dst_v, bufs, acc_sh, sems):
    c = lax.axis_index("c")
    s = lax.axis_index("s")
    wid = s * NC + c
    pltpu.sync_copy(zeros_hbm.at[pl.ds(s * RPT, RPT)],
                    acc_sh.at[pl.ds(s * RPT, RPT)])
    pltpu.sync_copy(src_hbm.at[wid], src_v)
    pltpu.sync_copy(dst_hbm.at[wid], dst_v)
    plsc.subcore_barrier()

    for b in range(NBUF):
        pltpu.async_copy(rows_hbm.at[src_v.at[b]], bufs.at[b], sems.at[b])

    def _wait_gather(b):
        pltpu.make_async_copy(rows_hbm.at[pl.ds(0, K)],
                              bufs.at[b], sems.at[b]).wait()

    def group(g, carry):
        for b in range(NBUF):
            j = g * NBUF + b
            _wait_gather(b)
            pltpu.sync_copy(bufs.at[b], acc_sh.at[dst_v.at[j]], add=True)
            pltpu.async_copy(rows_hbm.at[src_v.at[j + NBUF]],
                             bufs.at[b], sems.at[b])
        return carry
    lax.fori_loop(0, NGRP - 1, group, 0)

    for b in range(NBUF):
        j = (NGRP - 1) * NBUF + b
        _wait_gather(b)
        pltpu.sync_copy(bufs.at[b], acc_sh.at[dst_v.at[j]], add=True)

    plsc.subcore_barrier()
    pltpu.sync_copy(acc_sh.at[pl.ds(s * RPT, RPT)],
                    out_hbm.at[c].at[pl.ds(s * RPT, RPT)])


_sc_params = pltpu.CompilerParams(use_tc_tiling_on_sc=False)

_deg_call = pl.kernel(
    _deg_body,
    out_type=jax.ShapeDtypeStruct((NC, NP), jnp.float32),
    mesh=_sc_mesh(),
    compiler_params=_sc_params,
    scratch_types=[
        pltpu.VMEM((NCHUNK, K), jnp.int32),
        pltpu.VMEM((K,), jnp.float32),
        pltpu.VMEM_SHARED((NP,), jnp.float32),
    ],
)

_agg_call = pl.kernel(
    _agg_body,
    out_type=jax.ShapeDtypeStruct((NC, NP, HIDDEN), jnp.float32),
    mesh=_sc_mesh(),
    compiler_params=_sc_params,
    scratch_types=[
        pltpu.VMEM((NCHUNK, K), jnp.int32),
        pltpu.VMEM((NCHUNK, K), jnp.int32),
        pltpu.VMEM((NBUF, K, HIDDEN), jnp.float32),
        pltpu.VMEM_SHARED((NP, HIDDEN), jnp.float32),
        pltpu.SemaphoreType.DMA((NBUF,)),
    ],
)


def _tc1_body(xp, w1, degp, hs, dinvb):
    deg = degp[0, :] + degp[1, :] + 1.0
    dinv = lax.rsqrt(deg)
    db = jnp.broadcast_to(dinv[:, None], (NP, HIDDEN))
    dinvb[...] = db
    h = jnp.dot(xp[...], w1[...], preferred_element_type=jnp.float32)
    hs[...] = h * db


def _tc2_body(a1p, hs, dinvb, b1, gs):
    a1 = (a1p[0] + a1p[1] + hs[...]) * dinvb[...] + b1[...]
    gs[...] = jnp.maximum(a1, 0.0) * dinvb[...]


def _tc3_body(a2p, gs, dinvb, w2, b2, out):
    a2 = (a2p[0] + a2p[1] + gs[...]) * dinvb[...]
    out[...] = jnp.dot(a2, w2[...], preferred_element_type=jnp.float32) + b2[...]


_tc1 = pl.pallas_call(
    _tc1_body,
    out_shape=[jax.ShapeDtypeStruct((NP, HIDDEN), jnp.float32),
               jax.ShapeDtypeStruct((NP, HIDDEN), jnp.float32)],
)

_tc2 = pl.pallas_call(
    _tc2_body,
    out_shape=jax.ShapeDtypeStruct((NP, HIDDEN), jnp.float32),
)

_tc3 = pl.pallas_call(
    _tc3_body,
    out_shape=jax.ShapeDtypeStruct((NP, OUT_FEATS), jnp.float32),
)


def kernel(x, edge_index, W1, b1, W2, b2):
    src = edge_index[0].astype(jnp.int32).reshape(NW, NCHUNK, K)
    dst = edge_index[1].astype(jnp.int32).reshape(NW, NCHUNK, K)
    xp = jnp.zeros((NP, IN_FEATS), jnp.float32).at[:N_NODES].set(x)
    zeros1 = jnp.zeros((NP,), jnp.float32)
    zerosH = jnp.zeros((NP, HIDDEN), jnp.float32)
    onesK = jnp.ones((K,), jnp.float32)

    degp = _deg_call(dst, onesK, zeros1)
    hs, dinvb = _tc1(xp, W1, degp)
    a1p = _agg_call(hs, src, dst, zerosH)
    gs = _tc2(a1p, hs, dinvb, b1.reshape(1, HIDDEN))
    a2p = _agg_call(gs, src, dst, zerosH)
    out = _tc3(a2p, gs, dinvb, W2, b2.reshape(1, OUT_FEATS))
    return out[:N_NODES]


# speedup vs baseline: 63.7165x; 1.1343x over previous
import jax
import jax.numpy as jnp
from jax import lax
from jax.experimental import pallas as pl
from jax.experimental.pallas import tpu as pltpu
from jax.experimental.pallas import tpu_sc as plsc

N_NODES = 10000
N_EDGES = 320000
IN_FEATS = 128
HIDDEN = 16
OUT_FEATS = 2

NC, NS = 2, 16
NW = NC * NS
NP = 10240
RPT = NP // NS
EPW = N_EDGES // NW
K = 80
NCHUNK = EPW // K


def _sc_mesh():
    return plsc.VectorSubcoreMesh(
        core_axis_name="c", subcore_axis_name="s",
        num_cores=NC, num_subcores=NS)


def _deg_body(dst_hbm, ones_hbm, zeros_hbm, out_hbm, dst_v, ones_v, acc_sh,
              ssem):
    c = lax.axis_index("c")
    s = lax.axis_index("s")
    wid = s * NC + c
    pltpu.sync_copy(zeros_hbm.at[pl.ds(s * RPT, RPT)],
                    acc_sh.at[pl.ds(s * RPT, RPT)])
    pltpu.sync_copy(dst_hbm.at[wid], dst_v)
    pltpu.sync_copy(ones_hbm, ones_v)
    plsc.subcore_barrier()

    def fire(j, carry):
        pltpu.async_copy(ones_v, acc_sh.at[dst_v.at[j]], ssem, add=True)
        return carry
    lax.fori_loop(0, NCHUNK, fire, 0)

    def drain(j, carry):
        pltpu.make_async_copy(zeros_hbm.at[pl.ds(0, K)], ones_v, ssem).wait()
        return carry
    lax.fori_loop(0, NCHUNK, drain, 0)

    plsc.subcore_barrier()
    pltpu.sync_copy(acc_sh.at[pl.ds(s * RPT, RPT)],
                    out_hbm.at[c].at[pl.ds(s * RPT, RPT)])


SBC = 25
NSB = NCHUNK // SBC


def _agg_body(rows_hbm, src_hbm, dst_hbm, zeros_hbm, out_hbm,
              src_v, dst_v, bufs, acc_sh, gsems, ssems):
    c = lax.axis_index("c")
    s = lax.axis_index("s")
    wid = s * NC + c
    pltpu.sync_copy(zeros_hbm.at[pl.ds(s * RPT, RPT)],
                    acc_sh.at[pl.ds(s * RPT, RPT)])
    pltpu.sync_copy(src_hbm.at[wid], src_v)
    pltpu.sync_copy(dst_hbm.at[wid], dst_v)
    plsc.subcore_barrier()

    def fire_gathers(sb, bi):
        base = sb * SBC
        def f(j, carry):
            pltpu.async_copy(rows_hbm.at[src_v.at[base + j]],
                             bufs.at[bi].at[j], gsems.at[bi])
            return carry
        lax.fori_loop(0, SBC, f, 0)

    def fire_scatters(sb, bi):
        base = sb * SBC
        def f(j, carry):
            pltpu.async_copy(bufs.at[bi].at[j], acc_sh.at[dst_v.at[base + j]],
                             ssems.at[bi], add=True)
            return carry
        lax.fori_loop(0, SBC, f, 0)

    def drain(sem):
        def f(j, carry):
            pltpu.make_async_copy(rows_hbm.at[pl.ds(0, K)],
                                  bufs.at[0].at[0], sem).wait()
            return carry
        lax.fori_loop(0, SBC, f, 0)

    fire_gathers(0, 0)
    for sb in range(NSB):
        bi = sb % 2
        drain(gsems.at[bi])
        if sb >= 1:
            drain(ssems.at[(sb - 1) % 2])
        if sb + 1 < NSB:
            fire_gathers(sb + 1, (sb + 1) % 2)
        fire_scatters(sb, bi)
    drain(ssems.at[(NSB - 1) % 2])

    plsc.subcore_barrier()
    pltpu.sync_copy(acc_sh.at[pl.ds(s * RPT, RPT)],
                    out_hbm.at[c].at[pl.ds(s * RPT, RPT)])


_sc_params = pltpu.CompilerParams(use_tc_tiling_on_sc=False)

_deg_call = pl.kernel(
    _deg_body,
    out_type=jax.ShapeDtypeStruct((NC, NP), jnp.float32),
    mesh=_sc_mesh(),
    compiler_params=_sc_params,
    scratch_types=[
        pltpu.VMEM((NCHUNK, K), jnp.int32),
        pltpu.VMEM((K,), jnp.float32),
        pltpu.VMEM_SHARED((NP,), jnp.float32),
        pltpu.SemaphoreType.DMA,
    ],
)

_agg_call = pl.kernel(
    _agg_body,
    out_type=jax.ShapeDtypeStruct((NC, NP, HIDDEN), jnp.float32),
    mesh=_sc_mesh(),
    compiler_params=_sc_params,
    scratch_types=[
        pltpu.VMEM((NCHUNK, K), jnp.int32),
        pltpu.VMEM((NCHUNK, K), jnp.int32),
        pltpu.VMEM((2, SBC, K, HIDDEN), jnp.float32),
        pltpu.VMEM_SHARED((NP, HIDDEN), jnp.float32),
        pltpu.SemaphoreType.DMA((2,)),
        pltpu.SemaphoreType.DMA((2,)),
    ],
)


def _tc1_body(xp, w1, degp, hs, dinvb):
    deg = degp[0, :] + degp[1, :] + 1.0
    dinv = lax.rsqrt(deg)
    db = jnp.broadcast_to(dinv[:, None], (NP, HIDDEN))
    dinvb[...] = db
    h = jnp.dot(xp[...], w1[...], preferred_element_type=jnp.float32)
    hs[...] = h * db


def _tc2_body(a1p, hs, dinvb, b1, gs):
    a1 = (a1p[0] + a1p[1] + hs[...]) * dinvb[...] + b1[...]
    gs[...] = jnp.maximum(a1, 0.0) * dinvb[...]


def _tc3_body(a2p, gs, dinvb, w2, b2, out):
    a2 = (a2p[0] + a2p[1] + gs[...]) * dinvb[...]
    out[...] = jnp.dot(a2, w2[...], preferred_element_type=jnp.float32) + b2[...]


_tc1 = pl.pallas_call(
    _tc1_body,
    out_shape=[jax.ShapeDtypeStruct((NP, HIDDEN), jnp.float32),
               jax.ShapeDtypeStruct((NP, HIDDEN), jnp.float32)],
)

_tc2 = pl.pallas_call(
    _tc2_body,
    out_shape=jax.ShapeDtypeStruct((NP, HIDDEN), jnp.float32),
)

_tc3 = pl.pallas_call(
    _tc3_body,
    out_shape=jax.ShapeDtypeStruct((NP, OUT_FEATS), jnp.float32),
)


def kernel(x, edge_index, W1, b1, W2, b2):
    src = edge_index[0].astype(jnp.int32).reshape(NW, NCHUNK, K)
    dst = edge_index[1].astype(jnp.int32).reshape(NW, NCHUNK, K)
    xp = jnp.zeros((NP, IN_FEATS), jnp.float32).at[:N_NODES].set(x)
    zeros1 = jnp.zeros((NP,), jnp.float32)
    zerosH = jnp.zeros((NP, HIDDEN), jnp.float32)
    onesK = jnp.ones((K,), jnp.float32)

    degp = _deg_call(dst, onesK, zeros1)
    hs, dinvb = _tc1(xp, W1, degp)
    a1p = _agg_call(hs, src, dst, zerosH)
    gs = _tc2(a1p, hs, dinvb, b1.reshape(1, HIDDEN))
    a2p = _agg_call(gs, src, dst, zerosH)
    out = _tc3(a2p, gs, dinvb, W2, b2.reshape(1, OUT_FEATS))
    return out[:N_NODES]


# speedup vs baseline: 89.9084x; 1.4111x over previous
import jax
import jax.numpy as jnp
from jax import lax
from jax.experimental import pallas as pl
from jax.experimental.pallas import tpu as pltpu
from jax.experimental.pallas import tpu_sc as plsc

N_NODES = 10000
N_EDGES = 320000
IN_FEATS = 128
HIDDEN = 16
OUT_FEATS = 2

NC, NS = 2, 16
NW = NC * NS
NP = 10240
NQ = NP // 8
XQ = N_NODES // 8
RPT = NP // NS
EPW = N_EDGES // NW
K = 80
NCHUNK = EPW // K
SBC = 25
NSB = NCHUNK // SBC


def _sc_mesh():
    return plsc.VectorSubcoreMesh(
        core_axis_name="c", subcore_axis_name="s",
        num_cores=NC, num_subcores=NS)


def _deg_body(e_hbm, ones_hbm, zeros_hbm, out_hbm, dst_v, ones_v, acc_sh,
              ssem):
    c = lax.axis_index("c")
    s = lax.axis_index("s")
    wid = s * NC + c
    pltpu.sync_copy(zeros_hbm.at[pl.ds(s * RPT, RPT)],
                    acc_sh.at[pl.ds(s * RPT, RPT)])
    pltpu.sync_copy(e_hbm.at[1].at[wid], dst_v)
    pltpu.sync_copy(ones_hbm, ones_v)
    plsc.subcore_barrier()

    def fire(j, carry):
        pltpu.async_copy(ones_v, acc_sh.at[dst_v.at[j]], ssem, add=True)
        return carry
    lax.fori_loop(0, NCHUNK, fire, 0)

    def drain(j, carry):
        pltpu.make_async_copy(zeros_hbm.at[pl.ds(0, K)], ones_v, ssem).wait()
        return carry
    lax.fori_loop(0, NCHUNK, drain, 0)

    plsc.subcore_barrier()
    pltpu.sync_copy(acc_sh.at[pl.ds(s * RPT, RPT)],
                    out_hbm.at[c].at[pl.ds(s * RPT, RPT)])


def _agg_body(rows_hbm, e_hbm, zeros_hbm, out_hbm,
              src_v, dst_v, bufs, acc_sh, gsems, ssems):
    c = lax.axis_index("c")
    s = lax.axis_index("s")
    wid = s * NC + c
    pltpu.sync_copy(zeros_hbm.at[pl.ds(s * RPT, RPT)],
                    acc_sh.at[pl.ds(s * RPT, RPT)])
    pltpu.sync_copy(e_hbm.at[0].at[wid], src_v)
    pltpu.sync_copy(e_hbm.at[1].at[wid], dst_v)
    plsc.subcore_barrier()

    def fire_gathers(sb, bi):
        base = sb * SBC
        def f(j, carry):
            pltpu.async_copy(rows_hbm.at[src_v.at[base + j]],
                             bufs.at[bi].at[j], gsems.at[bi])
            return carry
        lax.fori_loop(0, SBC, f, 0)

    def fire_scatters(sb, bi):
        base = sb * SBC
        def f(j, carry):
            pltpu.async_copy(bufs.at[bi].at[j], acc_sh.at[dst_v.at[base + j]],
                             ssems.at[bi], add=True)
            return carry
        lax.fori_loop(0, SBC, f, 0)

    def drain(sem):
        def f(j, carry):
            pltpu.make_async_copy(rows_hbm.at[pl.ds(0, K)],
                                  bufs.at[0].at[0], sem).wait()
            return carry
        lax.fori_loop(0, SBC, f, 0)

    fire_gathers(0, 0)
    for sb in range(NSB):
        bi = sb % 2
        drain(gsems.at[bi])
        if sb >= 1:
            drain(ssems.at[(sb - 1) % 2])
        if sb + 1 < NSB:
            fire_gathers(sb + 1, (sb + 1) % 2)
        fire_scatters(sb, bi)
    drain(ssems.at[(NSB - 1) % 2])

    plsc.subcore_barrier()
    pltpu.sync_copy(acc_sh.at[pl.ds(s * RPT, RPT)],
                    out_hbm.at[c].at[pl.ds(s * RPT, RPT)])


_sc_params = pltpu.CompilerParams(use_tc_tiling_on_sc=False)

_deg_call = pl.kernel(
    _deg_body,
    out_type=jax.ShapeDtypeStruct((NC, NP, HIDDEN), jnp.float32),
    mesh=_sc_mesh(),
    compiler_params=_sc_params,
    scratch_types=[
        pltpu.VMEM((NCHUNK, K), jnp.int32),
        pltpu.VMEM((K, HIDDEN), jnp.float32),
        pltpu.VMEM_SHARED((NP, HIDDEN), jnp.float32),
        pltpu.SemaphoreType.DMA,
    ],
)

_agg_call = pl.kernel(
    _agg_body,
    out_type=jax.ShapeDtypeStruct((NC, NP, HIDDEN), jnp.float32),
    mesh=_sc_mesh(),
    compiler_params=_sc_params,
    scratch_types=[
        pltpu.VMEM((NCHUNK, K), jnp.int32),
        pltpu.VMEM((NCHUNK, K), jnp.int32),
        pltpu.VMEM((2, SBC, K, HIDDEN), jnp.float32),
        pltpu.VMEM_SHARED((NP, HIDDEN), jnp.float32),
        pltpu.SemaphoreType.DMA((2,)),
        pltpu.SemaphoreType.DMA((2,)),
    ],
)


def _tc1_body(xq, w1b, degq, hsq, dinvq):
    dinv = lax.rsqrt(degq[0] + degq[1] + 1.0)
    dinvq[...] = dinv
    h = jnp.dot(xq[...], w1b[...], preferred_element_type=jnp.float32)
    hsq[pl.ds(0, XQ), :] = h * dinv[0:XQ, :]


def _tc2_body(a1p, hsq, dinvq, b1t, gsq):
    a1 = (a1p[0] + a1p[1] + hsq[...]) * dinvq[...] + b1t[...]
    gsq[...] = jnp.maximum(a1, 0.0) * dinvq[...]


def _tc3_body(a2p, gsq, dinvq, w2b, b2t, outq):
    a2 = (a2p[0] + a2p[1] + gsq[...]) * dinvq[...]
    outq[...] = jnp.dot(a2, w2b[...], preferred_element_type=jnp.float32) \
        + b2t[...]


_tc1 = pl.pallas_call(
    _tc1_body,
    out_shape=[jax.ShapeDtypeStruct((NQ, 128), jnp.float32),
               jax.ShapeDtypeStruct((NQ, 128), jnp.float32)],
)

_tc2 = pl.pallas_call(
    _tc2_body,
    out_shape=jax.ShapeDtypeStruct((NQ, 128), jnp.float32),
)

_tc3 = pl.pallas_call(
    _tc3_body,
    out_shape=jax.ShapeDtypeStruct((NQ, 8 * OUT_FEATS), jnp.float32),
)


def kernel(x, edge_index, W1, b1, W2, b2):
    e = edge_index.astype(jnp.int32).reshape(2, NW, NCHUNK, K)
    xq = x.reshape(XQ, 8 * IN_FEATS)
    w1b = jnp.kron(jnp.eye(8, dtype=jnp.float32), W1)
    w2b = jnp.kron(jnp.eye(8, dtype=jnp.float32), W2)
    b1t = jnp.tile(b1, 8).reshape(1, 128)
    b2t = jnp.tile(b2, 8).reshape(1, 8 * OUT_FEATS)
    zerosH = jnp.zeros((NP, HIDDEN), jnp.float32)
    onesKH = jnp.ones((K, HIDDEN), jnp.float32)

    degp = _deg_call(e, onesKH, zerosH)
    hsq, dinvq = _tc1(xq, w1b, degp.reshape(NC, NQ, 128))
    a1p = _agg_call(hsq.reshape(NP, HIDDEN), e, zerosH)
    gsq = _tc2(a1p.reshape(NC, NQ, 128), hsq, dinvq, b1t)
    a2p = _agg_call(gsq.reshape(NP, HIDDEN), e, zerosH)
    outq = _tc3(a2p.reshape(NC, NQ, 128), gsq, dinvq, w2b, b2t)
    return outq.reshape(NP, OUT_FEATS)[:N_NODES]
